# Initial kernel scaffold; baseline (speedup 1.0000x reference)
import jax
import jax.numpy as jnp
from jax.experimental import pallas as pl


def kernel(mm_f_data_matrix, mm_f_edges, mm_s_data_matrix, mm_s_edges, mm_g_data_matrix, mm_g_edges, x_m, W_f1, b_f1, W_f2, b_f2, W_s1, b_s1, W_s2, b_s2, W_g1, b_g1, W_g2, b_g2, fc1_W, fc1_b, fc2_W, fc2_b, cnn_w, cnn_b):
    raise NotImplementedError("write your pallas kernel here")


# speedup vs baseline: 4.5053x; 4.5053x over previous
import functools

import jax
import jax.numpy as jnp
from jax import lax
from jax.experimental import pallas as pl
from jax.experimental.pallas import tpu as pltpu
from jax.experimental.pallas import tpu_sc as plsc

N = 10000
FM = 128
NP = 10240
E = 320000
EP = 327680
NW = 32
PT = EP // NW
CA = 1280
GA = CA // 128
NCA = PT // CA
CM = 256
GM = CM // 128
NCM = PT // CM
RPT = NP // 16
RB = 2048
NB = NP // RB


def _sc_mesh():
    return plsc.VectorSubcoreMesh(core_axis_name="c", subcore_axis_name="s")


def _sc_edge_prep(m1d, src1d, dst1d):
    @functools.partial(
        pl.kernel,
        mesh=_sc_mesh(),
        out_type=(jax.ShapeDtypeStruct((EP,), jnp.float32),
                  jax.ShapeDtypeStruct((2 * NP,), jnp.float32)),
        scratch_types=[
            pltpu.VMEM((GA, 128), jnp.int32),
            pltpu.VMEM((GA, 128), jnp.int32),
            pltpu.VMEM((GA, 128), jnp.int32),
            pltpu.VMEM((GA, 128), jnp.float32),
            pltpu.VMEM((RPT,), jnp.float32),
            pltpu.VMEM_SHARED((NP,), jnp.float32),
            pltpu.SemaphoreType.DMA,
        ],
    )
    def k(m_h, src_h, dst_h, ew_h, degp_h,
          sref, dref, fref, ewb, zbuf, deg_sh, sem):
        cid = lax.axis_index("c")
        sid = lax.axis_index("s")
        wid = cid * 16 + sid

        for i in range(RPT // 16):
            zbuf[pl.ds(i * 16, 16)] = jnp.zeros((16,), jnp.float32)
        pltpu.sync_copy(zbuf, deg_sh.at[pl.ds(sid * RPT, RPT)])
        plsc.subcore_barrier()

        def chunk(c, carry):
            base = wid * PT + c * CA
            for g in range(GA):
                pltpu.sync_copy(src_h.at[pl.ds(base + g * 128, 128)],
                                sref.at[g])
                pltpu.sync_copy(dst_h.at[pl.ds(base + g * 128, 128)],
                                dref.at[g])
            for g in range(GA):
                for o in range(8):
                    s16 = sref[g, pl.ds(o * 16, 16)]
                    d16 = dref[g, pl.ds(o * 16, 16)]
                    fref[g, pl.ds(o * 16, 16)] = s16 * N + d16
            cps = [pltpu.async_copy(m_h.at[fref.at[g]], ewb.at[g], sem)
                   for g in range(GA)]
            for cp in cps:
                cp.wait()
            for g in range(GA):
                pltpu.sync_copy(ewb.at[g],
                                ew_h.at[pl.ds(base + g * 128, 128)])
                pltpu.sync_copy(ewb.at[g], deg_sh.at[dref.at[g]], add=True)
            return carry

        lax.fori_loop(0, NCA, chunk, 0)
        plsc.subcore_barrier()
        pltpu.sync_copy(deg_sh.at[pl.ds(sid * RPT, RPT)], zbuf)
        pltpu.sync_copy(zbuf, degp_h.at[pl.ds(cid * NP + sid * RPT, RPT)])

    return k(m1d, src1d, dst1d)


def _sc_msg(y, src1d, dst1d, ew1d):
    @functools.partial(
        pl.kernel,
        mesh=_sc_mesh(),
        out_type=jax.ShapeDtypeStruct((2 * NP, FM), jnp.float32),
        scratch_types=[
            pltpu.VMEM((GM, 128), jnp.int32),
            pltpu.VMEM((GM, 128), jnp.int32),
            pltpu.VMEM((CM,), jnp.float32),
            pltpu.VMEM((CM, FM), jnp.float32),
            pltpu.VMEM_SHARED((NP, FM), jnp.float32),
            pltpu.SemaphoreType.DMA,
        ],
    )
    def k(y_h, src_h, dst_h, ew_h, msgp_h, sidx, didx, ewf, rows, acc_sh, sem):
        cid = lax.axis_index("c")
        sid = lax.axis_index("s")
        wid = cid * 16 + sid

        def zrow(r, carry):
            for kk in range(FM // 16):
                rows[r, pl.ds(kk * 16, 16)] = jnp.zeros((16,), jnp.float32)
            return carry

        lax.fori_loop(0, CM, zrow, 0)
        for p in range(RPT // 128):
            pltpu.sync_copy(rows.at[pl.ds(0, 128)],
                            acc_sh.at[pl.ds(sid * RPT + p * 128, 128)])
        plsc.subcore_barrier()

        def chunk(c, carry):
            eb = wid * PT + c * CM
            for g in range(GM):
                pltpu.sync_copy(src_h.at[pl.ds(eb + g * 128, 128)],
                                sidx.at[g])
                pltpu.sync_copy(dst_h.at[pl.ds(eb + g * 128, 128)],
                                didx.at[g])
            pltpu.sync_copy(ew_h.at[pl.ds(eb, CM)], ewf)
            cps = [pltpu.async_copy(y_h.at[sidx.at[g]],
                                    rows.at[pl.ds(g * 128, 128)], sem)
                   for g in range(GM)]
            for cp in cps:
                cp.wait()

            def scale(j, carry2):
                w16 = ewf[pl.ds(j * 16, 16)]
                for i in range(16):
                    e = j * 16 + i
                    w = jnp.full((16,), w16[i])
                    for kk in range(FM // 16):
                        rows[e, pl.ds(kk * 16, 16)] = (
                            rows[e, pl.ds(kk * 16, 16)] * w)
                return carry2

            lax.fori_loop(0, CM // 16, scale, 0)
            for g in range(GM):
                pltpu.sync_copy(rows.at[pl.ds(g * 128, 128)],
                                acc_sh.at[didx.at[g]], add=True)
            return carry

        lax.fori_loop(0, NCM, chunk, 0)
        plsc.subcore_barrier()
        for p in range(RPT // 128):
            pltpu.sync_copy(acc_sh.at[pl.ds(sid * RPT + p * 128, 128)],
                            rows.at[pl.ds(0, 128)])
            pltpu.sync_copy(rows.at[pl.ds(0, 128)],
                            msgp_h.at[pl.ds(cid * NP + sid * RPT + p * 128,
                                            128)])

    return k(y, src1d, dst1d, ew1d)


def _tc_prep(xp, dega8, W1s):
    def body(x_ref, deg_ref, w_ref, yf_ref, ys_ref, yg_ref, dis_ref):
        x = x_ref[...]
        dis_cols = []
        for v, y_ref in enumerate((yf_ref, ys_ref, yg_ref)):
            deg = deg_ref[:, 2 * v:2 * v + 1] + deg_ref[:, 2 * v + 1:2 * v + 2] + 1.0
            dis = jnp.where(deg > 0, lax.rsqrt(jnp.where(deg > 0, deg, 1.0)), 0.0)
            xw = jnp.dot(x, w_ref[v], preferred_element_type=jnp.float32)
            y_ref[...] = dis * xw
            dis_cols.append(dis)
        dis_ref[...] = jnp.concatenate(
            dis_cols + [jnp.zeros((RB, 5), jnp.float32)], axis=1)

    return pl.pallas_call(
        body,
        grid=(NB,),
        in_specs=[
            pl.BlockSpec((RB, FM), lambda i: (i, 0)),
            pl.BlockSpec((RB, 8), lambda i: (i, 0)),
            pl.BlockSpec((3, FM, FM), lambda i: (0, 0, 0)),
        ],
        out_specs=[
            pl.BlockSpec((RB, FM), lambda i: (i, 0)),
            pl.BlockSpec((RB, FM), lambda i: (i, 0)),
            pl.BlockSpec((RB, FM), lambda i: (i, 0)),
            pl.BlockSpec((RB, 8), lambda i: (i, 0)),
        ],
        out_shape=[jax.ShapeDtypeStruct((NP, FM), jnp.float32)] * 3
        + [jax.ShapeDtypeStruct((NP, 8), jnp.float32)],
    )(xp, dega8, W1s)


def _tc_combine(msgps, ys, dis8, bs, W2s=None):
    with_w2 = W2s is not None

    def body(*refs):
        (mf, ms, mg, yf, ys_, yg, dis_ref, b_ref), rest = refs[:8], refs[8:]
        if with_w2:
            w2_ref = rest[0]
            h_refs = rest[1:4]
            y2_refs = rest[4:7]
            ps_ref = rest[7]
        else:
            h_refs = rest[0:3]
            y2_refs = None
            ps_ref = rest[3]
        gid = pl.program_id(0)
        gr = lax.broadcasted_iota(jnp.int32, (RB, FM), 0) + gid * RB
        for v, (m_ref, y_ref) in enumerate(((mf, yf), (ms, ys_), (mg, yg))):
            dis = dis_ref[:, v:v + 1]
            t = m_ref[0] + m_ref[1] + y_ref[...]
            h = jnp.maximum(dis * t + b_ref[v][None, :], 0.0)
            h_refs[v][...] = h
            hm = jnp.where(gr < N, h, 0.0)
            s = jnp.sum(hm, axis=0, keepdims=True)
            ps_ref[v, 0] = jnp.concatenate(
                [s, jnp.zeros((7, FM), jnp.float32)], axis=0)
            if with_w2:
                y2_refs[v][...] = dis * jnp.dot(
                    h, w2_ref[v], preferred_element_type=jnp.float32)

    blk = pl.BlockSpec((RB, FM), lambda i: (i, 0))
    mblk = pl.BlockSpec((2, RB, FM), lambda i: (0, i, 0))
    in_specs = [mblk, mblk, mblk, blk, blk, blk,
                pl.BlockSpec((RB, 8), lambda i: (i, 0)),
                pl.BlockSpec((3, FM), lambda i: (0, 0))]
    out_specs = [blk, blk, blk]
    out_shape = [jax.ShapeDtypeStruct((NP, FM), jnp.float32)] * 3
    if with_w2:
        in_specs.append(pl.BlockSpec((3, FM, FM), lambda i: (0, 0, 0)))
        out_specs += [blk, blk, blk]
        out_shape += [jax.ShapeDtypeStruct((NP, FM), jnp.float32)] * 3
    out_specs.append(pl.BlockSpec((3, 1, 8, FM), lambda i: (0, i, 0, 0)))
    out_shape.append(jax.ShapeDtypeStruct((3, NB, 8, FM), jnp.float32))

    args = list(msgps) + list(ys) + [dis8, bs] + ([W2s] if with_w2 else [])
    return pl.pallas_call(
        body, grid=(NB,), in_specs=in_specs, out_specs=out_specs,
        out_shape=out_shape,
    )(*args)


def _tc_att(pt8, fc1_W, fc1_b2, fc2_W, fc2_b2, cnn_w2):
    def body(p_ref, w1_ref, b1_ref, w2_ref, b2_ref, cw_ref, out_ref):
        m = jnp.sum(p_ref[...], axis=0, keepdims=True) / float(N * FM)
        a1 = jnp.maximum(
            jnp.dot(m[:, :6], w1_ref[...], preferred_element_type=jnp.float32)
            + b1_ref[...], 0.0)
        a2 = jax.nn.sigmoid(
            jnp.dot(a1, w2_ref[...], preferred_element_type=jnp.float32)
            + b2_ref[...])
        out_ref[...] = a2 * cw_ref[...]

    return pl.pallas_call(
        body,
        out_shape=jax.ShapeDtypeStruct((1, 6), jnp.float32),
    )(pt8, fc1_W, fc1_b2, fc2_W, fc2_b2, cnn_w2)


def _tc_mix(h1s, h2s, w8):
    def body(h1f, h1s_, h1g, h2f, h2s_, h2g, w_ref, out_ref):
        acc = (h1f[...] * w_ref[0, 0] + h2f[...] * w_ref[0, 1]
               + h1s_[...] * w_ref[0, 2] + h2s_[...] * w_ref[0, 3]
               + h1g[...] * w_ref[0, 4] + h2g[...] * w_ref[0, 5]
               + w_ref[0, 6])
        out_ref[...] = acc

    blk = pl.BlockSpec((RB, FM), lambda i: (i, 0))
    return pl.pallas_call(
        body,
        grid=(NB,),
        in_specs=[blk] * 6 + [pl.BlockSpec(memory_space=pltpu.SMEM)],
        out_specs=blk,
        out_shape=jax.ShapeDtypeStruct((NP, FM), jnp.float32),
    )(*h1s, *h2s, w8)


def kernel(mm_f_data_matrix, mm_f_edges, mm_s_data_matrix, mm_s_edges,
           mm_g_data_matrix, mm_g_edges, x_m, W_f1, b_f1, W_f2, b_f2,
           W_s1, b_s1, W_s2, b_s2, W_g1, b_g1, W_g2, b_g2,
           fc1_W, fc1_b, fc2_W, fc2_b, cnn_w, cnn_b):
    xp = jnp.pad(x_m, ((0, NP - N), (0, 0)))
    pad_src = jnp.zeros((EP - E,), jnp.int32)
    pad_dst = jnp.full((EP - E,), N, jnp.int32)

    srcs, dsts, ew1ds, degs = [], [], [], []
    for M, e in ((mm_f_data_matrix, mm_f_edges),
                 (mm_s_data_matrix, mm_s_edges),
                 (mm_g_data_matrix, mm_g_edges)):
        src1d = jnp.concatenate([e[0], pad_src])
        dst1d = jnp.concatenate([e[1], pad_dst])
        ew1d, degp = _sc_edge_prep(M.reshape(N * N), src1d, dst1d)
        srcs.append(src1d)
        dsts.append(dst1d)
        ew1ds.append(ew1d)
        degs.append(degp.reshape(2, NP))

    dega8 = jnp.pad(jnp.concatenate(degs, axis=0).T, ((0, 0), (0, 2)))
    W1s = jnp.stack([W_f1, W_s1, W_g1])
    b1s = jnp.stack([b_f1, b_s1, b_g1])
    W2s = jnp.stack([W_f2, W_s2, W_g2])
    b2s = jnp.stack([b_f2, b_s2, b_g2])

    y1f, y1s, y1g, dis8 = _tc_prep(xp, dega8, W1s)

    msg1 = [_sc_msg(y, srcs[v], dsts[v], ew1ds[v]).reshape(2, NP, FM)
            for v, y in enumerate((y1f, y1s, y1g))]
    h1f, h1s, h1g, y2f, y2s, y2g, ps1 = _tc_combine(
        msg1, (y1f, y1s, y1g), dis8, b1s, W2s)

    msg2 = [_sc_msg(y, srcs[v], dsts[v], ew1ds[v]).reshape(2, NP, FM)
            for v, y in enumerate((y2f, y2s, y2g))]
    h2f, h2s, h2g, ps2 = _tc_combine(msg2, (y2f, y2s, y2g), dis8, b2s)

    pt = jnp.stack([ps1[0], ps2[0], ps1[1], ps2[1], ps1[2], ps2[2]])
    pt8 = jnp.pad(pt.reshape(6, NB * 8 * FM).T, ((0, 0), (0, 2)))
    w6 = _tc_att(pt8, fc1_W, fc1_b.reshape(1, 30), fc2_W,
                 fc2_b.reshape(1, 6), cnn_w.reshape(1, 6))
    w8 = jnp.concatenate(
        [w6, cnn_b.reshape(1, 1), jnp.zeros((1, 1), jnp.float32)], axis=1)

    out = _tc_mix((h1f, h1s, h1g), (h2f, h2s, h2g), w8)
    return out[:N]


# speedup vs baseline: 5.9310x; 1.3165x over previous
import functools

import jax
import jax.numpy as jnp
from jax import lax
from jax.experimental import pallas as pl
from jax.experimental.pallas import tpu as pltpu
from jax.experimental.pallas import tpu_sc as plsc

N = 10000
FM = 128
NP = 10240
E = 320000
EP = 327680
NW = 32
PT = EP // NW
CA = 1024
GA = CA // 128
NCA = PT // CA
CM = 128
NCM = PT // CM
RPT = NP // 16
RB = 2048
NB = NP // RB


def _sc_mesh():
    return plsc.VectorSubcoreMesh(core_axis_name="c", subcore_axis_name="s")


def _sc_edge_prep(m1d, src2d, dst2d):
    @functools.partial(
        pl.kernel,
        mesh=_sc_mesh(),
        out_type=(jax.ShapeDtypeStruct((EP // 128, 128), jnp.float32),
                  jax.ShapeDtypeStruct((2 * NP,), jnp.float32)),
        scratch_types=[
            pltpu.VMEM((GA, 128), jnp.int32),
            pltpu.VMEM((GA, 128), jnp.int32),
            pltpu.VMEM((GA, 128), jnp.int32),
            pltpu.VMEM((GA, 128), jnp.int32),
            pltpu.VMEM((GA, 128), jnp.float32),
            pltpu.VMEM((RPT,), jnp.float32),
            pltpu.VMEM_SHARED((NP,), jnp.float32),
            pltpu.SemaphoreType.DMA,
            pltpu.SemaphoreType.DMA,
        ],
    )
    def k(m_h, src_h, dst_h, ew_h, degp_h,
          sref, dref, fref, dsc, ewb, zbuf, deg_sh, sem_g, sem_sc):
        cid = lax.axis_index("c")
        sid = lax.axis_index("s")
        wid = cid * 16 + sid

        for i in range(RPT // 16):
            zbuf[pl.ds(i * 16, 16)] = jnp.zeros((16,), jnp.float32)
        pltpu.sync_copy(zbuf, deg_sh.at[pl.ds(sid * RPT, RPT)])
        plsc.subcore_barrier()

        def chunk(c, carry):
            gb = wid * (PT // 128) + c * GA
            pltpu.sync_copy(src_h.at[pl.ds(gb, GA)], sref)
            pltpu.sync_copy(dst_h.at[pl.ds(gb, GA)], dref)

            @pl.when(c > 0)
            def _():
                for g in range(GA):
                    pltpu.make_async_copy(
                        ewb.at[g], deg_sh.at[dsc.at[g]], sem_sc).wait()

            for g in range(GA):
                for o in range(8):
                    s16 = sref[g, pl.ds(o * 16, 16)]
                    d16 = dref[g, pl.ds(o * 16, 16)]
                    fref[g, pl.ds(o * 16, 16)] = s16 * N + d16
                    dsc[g, pl.ds(o * 16, 16)] = d16

            cps = [pltpu.async_copy(m_h.at[fref.at[g]], ewb.at[g], sem_g)
                   for g in range(GA)]
            for cp in cps:
                cp.wait()
            pltpu.sync_copy(ewb, ew_h.at[pl.ds(gb, GA)])
            for g in range(GA):
                pltpu.async_copy(ewb.at[g], deg_sh.at[dsc.at[g]], sem_sc,
                                 add=True)
            return carry

        lax.fori_loop(0, NCA, chunk, 0)
        for g in range(GA):
            pltpu.make_async_copy(ewb.at[g], deg_sh.at[dsc.at[g]],
                                  sem_sc).wait()
        plsc.subcore_barrier()
        pltpu.sync_copy(deg_sh.at[pl.ds(sid * RPT, RPT)], zbuf)
        pltpu.sync_copy(zbuf, degp_h.at[pl.ds(cid * NP + sid * RPT, RPT)])

    return k(m1d, src2d, dst2d)


def _sc_msg(y, src1d, dst1d, ew1d):
    @functools.partial(
        pl.kernel,
        mesh=_sc_mesh(),
        out_type=jax.ShapeDtypeStruct((2 * NP, FM), jnp.float32),
        scratch_types=[
            pltpu.VMEM((CM,), jnp.int32),
            pltpu.VMEM((CM,), jnp.int32),
            pltpu.VMEM((CM,), jnp.float32),
            pltpu.VMEM((CM, FM), jnp.float32),
            pltpu.VMEM((CM,), jnp.int32),
            pltpu.VMEM((CM,), jnp.int32),
            pltpu.VMEM((CM,), jnp.float32),
            pltpu.VMEM((CM, FM), jnp.float32),
            pltpu.VMEM_SHARED((NP, FM), jnp.float32),
            pltpu.SemaphoreType.DMA,
            pltpu.SemaphoreType.DMA,
            pltpu.SemaphoreType.DMA,
            pltpu.SemaphoreType.DMA,
        ],
    )
    def k(y_h, src_h, dst_h, ew_h, msgp_h,
          sidx0, didx0, ewf0, rows0, sidx1, didx1, ewf1, rows1,
          acc_sh, sem_i0, sem_i1, sem_g0, sem_g1):
        cid = lax.axis_index("c")
        sid = lax.axis_index("s")
        wid = cid * 16 + sid
        bufs = ((sidx0, didx0, ewf0, rows0, sem_i0, sem_g0),
                (sidx1, didx1, ewf1, rows1, sem_i1, sem_g1))

        def idx_start(c, b):
            sidx, didx, ewf, _, sem_i, _ = bufs[b]
            eb = wid * PT + c * CM
            pltpu.async_copy(src_h.at[pl.ds(eb, CM)], sidx, sem_i)
            pltpu.async_copy(dst_h.at[pl.ds(eb, CM)], didx, sem_i)
            pltpu.async_copy(ew_h.at[pl.ds(eb, CM)], ewf, sem_i)

        def idx_drain(b):
            sidx, didx, ewf, _, sem_i, _ = bufs[b]
            pltpu.make_async_copy(src_h.at[pl.ds(0, CM)], sidx, sem_i).wait()
            pltpu.make_async_copy(dst_h.at[pl.ds(0, CM)], didx, sem_i).wait()
            pltpu.make_async_copy(ew_h.at[pl.ds(0, CM)], ewf, sem_i).wait()

        def gather_start(b):
            sidx, _, _, rows, _, sem_g = bufs[b]
            pltpu.async_copy(y_h.at[sidx], rows, sem_g)

        def gather_drain(b):
            sidx, _, _, rows, _, sem_g = bufs[b]
            pltpu.make_async_copy(y_h.at[sidx], rows, sem_g).wait()

        def scale(b):
            _, _, ewf, rows, _, _ = bufs[b]

            def body(j, carry2):
                w16 = ewf[pl.ds(j * 16, 16)]
                for i in range(16):
                    e = j * 16 + i
                    w = jnp.full((16,), w16[i])
                    for kk in range(FM // 16):
                        rows[e, pl.ds(kk * 16, 16)] = (
                            rows[e, pl.ds(kk * 16, 16)] * w)
                return carry2

            lax.fori_loop(0, CM // 16, body, 0)

        def scatter(b):
            _, didx, _, rows, _, _ = bufs[b]
            pltpu.sync_copy(rows, acc_sh.at[didx], add=True)

        def zrow(r, carry):
            for kk in range(FM // 16):
                rows0[r, pl.ds(kk * 16, 16)] = jnp.zeros((16,), jnp.float32)
            return carry

        lax.fori_loop(0, CM, zrow, 0)
        for p in range(RPT // CM):
            pltpu.sync_copy(rows0,
                            acc_sh.at[pl.ds(sid * RPT + p * CM, CM)])
        plsc.subcore_barrier()

        idx_start(0, 0)
        idx_drain(0)
        gather_start(0)
        idx_start(1, 1)

        def step(i, carry):
            c0 = 2 * i
            c1 = c0 + 1
            idx_drain(1)
            gather_drain(0)
            gather_start(1)
            scale(0)
            scatter(0)

            @pl.when(c0 + 2 < NCM)
            def _():
                idx_start(c0 + 2, 0)

            @pl.when(c1 + 1 < NCM)
            def _():
                idx_drain(0)
                gather_start(0)

            gather_drain(1)
            scale(1)
            scatter(1)

            @pl.when(c1 + 2 < NCM)
            def _():
                idx_start(c1 + 2, 1)

            return carry

        lax.fori_loop(0, NCM // 2, step, 0)
        plsc.subcore_barrier()
        for p in range(RPT // CM):
            pltpu.sync_copy(acc_sh.at[pl.ds(sid * RPT + p * CM, CM)], rows0)
            pltpu.sync_copy(rows0,
                            msgp_h.at[pl.ds(cid * NP + sid * RPT + p * CM,
                                            CM)])

    return k(y, src1d, dst1d, ew1d)


def _tc_prep(xp, dega8, W1s):
    def body(x_ref, deg_ref, w_ref, yf_ref, ys_ref, yg_ref, dis_ref):
        x = x_ref[...]
        dis_cols = []
        for v, y_ref in enumerate((yf_ref, ys_ref, yg_ref)):
            deg = deg_ref[:, 2 * v:2 * v + 1] + deg_ref[:, 2 * v + 1:2 * v + 2] + 1.0
            dis = jnp.where(deg > 0, lax.rsqrt(jnp.where(deg > 0, deg, 1.0)), 0.0)
            xw = jnp.dot(x, w_ref[v], preferred_element_type=jnp.float32)
            y_ref[...] = dis * xw
            dis_cols.append(dis)
        dis_ref[...] = jnp.concatenate(
            dis_cols + [jnp.zeros((RB, 5), jnp.float32)], axis=1)

    return pl.pallas_call(
        body,
        grid=(NB,),
        in_specs=[
            pl.BlockSpec((RB, FM), lambda i: (i, 0)),
            pl.BlockSpec((RB, 8), lambda i: (i, 0)),
            pl.BlockSpec((3, FM, FM), lambda i: (0, 0, 0)),
        ],
        out_specs=[
            pl.BlockSpec((RB, FM), lambda i: (i, 0)),
            pl.BlockSpec((RB, FM), lambda i: (i, 0)),
            pl.BlockSpec((RB, FM), lambda i: (i, 0)),
            pl.BlockSpec((RB, 8), lambda i: (i, 0)),
        ],
        out_shape=[jax.ShapeDtypeStruct((NP, FM), jnp.float32)] * 3
        + [jax.ShapeDtypeStruct((NP, 8), jnp.float32)],
    )(xp, dega8, W1s)


def _tc_combine(msgps, ys, dis8, bs, W2s=None):
    with_w2 = W2s is not None

    def body(*refs):
        (mf, ms, mg, yf, ys_, yg, dis_ref, b_ref), rest = refs[:8], refs[8:]
        if with_w2:
            w2_ref = rest[0]
            h_refs = rest[1:4]
            y2_refs = rest[4:7]
            ps_ref = rest[7]
        else:
            h_refs = rest[0:3]
            y2_refs = None
            ps_ref = rest[3]
        gid = pl.program_id(0)
        gr = lax.broadcasted_iota(jnp.int32, (RB, FM), 0) + gid * RB
        for v, (m_ref, y_ref) in enumerate(((mf, yf), (ms, ys_), (mg, yg))):
            dis = dis_ref[:, v:v + 1]
            t = m_ref[0] + m_ref[1] + y_ref[...]
            h = jnp.maximum(dis * t + b_ref[v][None, :], 0.0)
            h_refs[v][...] = h
            hm = jnp.where(gr < N, h, 0.0)
            s = jnp.sum(hm, axis=0, keepdims=True)
            ps_ref[v, 0] = jnp.concatenate(
                [s, jnp.zeros((7, FM), jnp.float32)], axis=0)
            if with_w2:
                y2_refs[v][...] = dis * jnp.dot(
                    h, w2_ref[v], preferred_element_type=jnp.float32)

    blk = pl.BlockSpec((RB, FM), lambda i: (i, 0))
    mblk = pl.BlockSpec((2, RB, FM), lambda i: (0, i, 0))
    in_specs = [mblk, mblk, mblk, blk, blk, blk,
                pl.BlockSpec((RB, 8), lambda i: (i, 0)),
                pl.BlockSpec((3, FM), lambda i: (0, 0))]
    out_specs = [blk, blk, blk]
    out_shape = [jax.ShapeDtypeStruct((NP, FM), jnp.float32)] * 3
    if with_w2:
        in_specs.append(pl.BlockSpec((3, FM, FM), lambda i: (0, 0, 0)))
        out_specs += [blk, blk, blk]
        out_shape += [jax.ShapeDtypeStruct((NP, FM), jnp.float32)] * 3
    out_specs.append(pl.BlockSpec((3, 1, 8, FM), lambda i: (0, i, 0, 0)))
    out_shape.append(jax.ShapeDtypeStruct((3, NB, 8, FM), jnp.float32))

    args = list(msgps) + list(ys) + [dis8, bs] + ([W2s] if with_w2 else [])
    return pl.pallas_call(
        body, grid=(NB,), in_specs=in_specs, out_specs=out_specs,
        out_shape=out_shape,
    )(*args)


def _tc_att(pt8, fc1_W, fc1_b2, fc2_W, fc2_b2, cnn_w2):
    def body(p_ref, w1_ref, b1_ref, w2_ref, b2_ref, cw_ref, out_ref):
        m = jnp.sum(p_ref[...], axis=0, keepdims=True) / float(N * FM)
        a1 = jnp.maximum(
            jnp.dot(m[:, :6], w1_ref[...], preferred_element_type=jnp.float32)
            + b1_ref[...], 0.0)
        a2 = jax.nn.sigmoid(
            jnp.dot(a1, w2_ref[...], preferred_element_type=jnp.float32)
            + b2_ref[...])
        out_ref[...] = a2 * cw_ref[...]

    return pl.pallas_call(
        body,
        out_shape=jax.ShapeDtypeStruct((1, 6), jnp.float32),
    )(pt8, fc1_W, fc1_b2, fc2_W, fc2_b2, cnn_w2)


def _tc_mix(h1s, h2s, w8):
    def body(h1f, h1s_, h1g, h2f, h2s_, h2g, w_ref, out_ref):
        acc = (h1f[...] * w_ref[0, 0] + h2f[...] * w_ref[0, 1]
               + h1s_[...] * w_ref[0, 2] + h2s_[...] * w_ref[0, 3]
               + h1g[...] * w_ref[0, 4] + h2g[...] * w_ref[0, 5]
               + w_ref[0, 6])
        out_ref[...] = acc

    blk = pl.BlockSpec((RB, FM), lambda i: (i, 0))
    return pl.pallas_call(
        body,
        grid=(NB,),
        in_specs=[blk] * 6 + [pl.BlockSpec(memory_space=pltpu.SMEM)],
        out_specs=blk,
        out_shape=jax.ShapeDtypeStruct((NP, FM), jnp.float32),
    )(*h1s, *h2s, w8)


def kernel(mm_f_data_matrix, mm_f_edges, mm_s_data_matrix, mm_s_edges,
           mm_g_data_matrix, mm_g_edges, x_m, W_f1, b_f1, W_f2, b_f2,
           W_s1, b_s1, W_s2, b_s2, W_g1, b_g1, W_g2, b_g2,
           fc1_W, fc1_b, fc2_W, fc2_b, cnn_w, cnn_b):
    xp = jnp.pad(x_m, ((0, NP - N), (0, 0)))
    pad_src = jnp.zeros((EP - E,), jnp.int32)
    pad_dst = jnp.full((EP - E,), N, jnp.int32)

    srcs, dsts, ew1ds, degs = [], [], [], []
    for M, e in ((mm_f_data_matrix, mm_f_edges),
                 (mm_s_data_matrix, mm_s_edges),
                 (mm_g_data_matrix, mm_g_edges)):
        src1d = jnp.concatenate([e[0], pad_src])
        dst1d = jnp.concatenate([e[1], pad_dst])
        ew2d, degp = _sc_edge_prep(M.reshape(N * N),
                                   src1d.reshape(EP // 128, 128),
                                   dst1d.reshape(EP // 128, 128))
        srcs.append(src1d)
        dsts.append(dst1d)
        ew1ds.append(ew2d.reshape(EP))
        degs.append(degp.reshape(2, NP))

    dega8 = jnp.pad(jnp.concatenate(degs, axis=0).T, ((0, 0), (0, 2)))
    W1s = jnp.stack([W_f1, W_s1, W_g1])
    b1s = jnp.stack([b_f1, b_s1, b_g1])
    W2s = jnp.stack([W_f2, W_s2, W_g2])
    b2s = jnp.stack([b_f2, b_s2, b_g2])

    y1f, y1s, y1g, dis8 = _tc_prep(xp, dega8, W1s)

    msg1 = [_sc_msg(y, srcs[v], dsts[v], ew1ds[v]).reshape(2, NP, FM)
            for v, y in enumerate((y1f, y1s, y1g))]
    h1f, h1s, h1g, y2f, y2s, y2g, ps1 = _tc_combine(
        msg1, (y1f, y1s, y1g), dis8, b1s, W2s)

    msg2 = [_sc_msg(y, srcs[v], dsts[v], ew1ds[v]).reshape(2, NP, FM)
            for v, y in enumerate((y2f, y2s, y2g))]
    h2f, h2s, h2g, ps2 = _tc_combine(msg2, (y2f, y2s, y2g), dis8, b2s)

    pt = jnp.stack([ps1[0], ps2[0], ps1[1], ps2[1], ps1[2], ps2[2]])
    pt8 = jnp.pad(pt.reshape(6, NB * 8 * FM).T, ((0, 0), (0, 2)))
    w6 = _tc_att(pt8, fc1_W, fc1_b.reshape(1, 30), fc2_W,
                 fc2_b.reshape(1, 6), cnn_w.reshape(1, 6))
    w8 = jnp.concatenate(
        [w6, cnn_b.reshape(1, 1), jnp.zeros((1, 1), jnp.float32)], axis=1)

    out = _tc_mix((h1f, h1s, h1g), (h2f, h2s, h2g), w8)
    return out[:N]


# speedup vs baseline: 6.0103x; 1.0134x over previous
import functools

import jax
import jax.numpy as jnp
from jax import lax
from jax.experimental import pallas as pl
from jax.experimental.pallas import tpu as pltpu
from jax.experimental.pallas import tpu_sc as plsc

N = 10000
FM = 128
NP = 10240
E = 320000
EP = 327680
NW = 32
PT = EP // NW
CA = 1024
GA = CA // 128
NCA = PT // CA
CM = 128
NCM = PT // CM
RPT = NP // 16
RB = 2048
NB = NP // RB


def _sc_mesh():
    return plsc.VectorSubcoreMesh(core_axis_name="c", subcore_axis_name="s")


def _sc_edge_prep(m1d, src2d, dst2d):
    @functools.partial(
        pl.kernel,
        mesh=_sc_mesh(),
        out_type=(jax.ShapeDtypeStruct((EP // 128, 128), jnp.float32),
                  jax.ShapeDtypeStruct((2 * NP,), jnp.float32)),
        scratch_types=[
            pltpu.VMEM((GA, 128), jnp.int32),
            pltpu.VMEM((GA, 128), jnp.int32),
            pltpu.VMEM((GA, 128), jnp.int32),
            pltpu.VMEM((GA, 128), jnp.int32),
            pltpu.VMEM((GA, 128), jnp.float32),
            pltpu.VMEM((RPT,), jnp.float32),
            pltpu.VMEM_SHARED((NP,), jnp.float32),
            pltpu.SemaphoreType.DMA,
            pltpu.SemaphoreType.DMA,
        ],
    )
    def k(m_h, src_h, dst_h, ew_h, degp_h,
          sref, dref, fref, dsc, ewb, zbuf, deg_sh, sem_g, sem_sc):
        cid = lax.axis_index("c")
        sid = lax.axis_index("s")
        wid = cid * 16 + sid

        for i in range(RPT // 16):
            zbuf[pl.ds(i * 16, 16)] = jnp.zeros((16,), jnp.float32)
        pltpu.sync_copy(zbuf, deg_sh.at[pl.ds(sid * RPT, RPT)])
        plsc.subcore_barrier()

        def chunk(c, carry):
            gb = wid * (PT // 128) + c * GA
            pltpu.sync_copy(src_h.at[pl.ds(gb, GA)], sref)
            pltpu.sync_copy(dst_h.at[pl.ds(gb, GA)], dref)

            @pl.when(c > 0)
            def _():
                for g in range(GA):
                    pltpu.make_async_copy(
                        ewb.at[g], deg_sh.at[dsc.at[g]], sem_sc).wait()

            for g in range(GA):
                for o in range(8):
                    s16 = sref[g, pl.ds(o * 16, 16)]
                    d16 = dref[g, pl.ds(o * 16, 16)]
                    fref[g, pl.ds(o * 16, 16)] = s16 * N + d16
                    dsc[g, pl.ds(o * 16, 16)] = d16

            cps = [pltpu.async_copy(m_h.at[fref.at[g]], ewb.at[g], sem_g)
                   for g in range(GA)]
            for cp in cps:
                cp.wait()
            pltpu.sync_copy(ewb, ew_h.at[pl.ds(gb, GA)])
            for g in range(GA):
                pltpu.async_copy(ewb.at[g], deg_sh.at[dsc.at[g]], sem_sc,
                                 add=True)
            return carry

        lax.fori_loop(0, NCA, chunk, 0)
        for g in range(GA):
            pltpu.make_async_copy(ewb.at[g], deg_sh.at[dsc.at[g]],
                                  sem_sc).wait()
        plsc.subcore_barrier()
        pltpu.sync_copy(deg_sh.at[pl.ds(sid * RPT, RPT)], zbuf)
        pltpu.sync_copy(zbuf, degp_h.at[pl.ds(cid * NP + sid * RPT, RPT)])

    return k(m1d, src2d, dst2d)


def _sc_msg(y, src1d, dst1d, ew1d):
    @functools.partial(
        pl.kernel,
        mesh=_sc_mesh(),
        out_type=jax.ShapeDtypeStruct((2 * NP, FM), jnp.float32),
        scratch_types=[
            pltpu.VMEM((CM,), jnp.int32),
            pltpu.VMEM((CM,), jnp.int32),
            pltpu.VMEM((CM,), jnp.float32),
            pltpu.VMEM((CM, FM), jnp.float32),
            pltpu.VMEM((CM,), jnp.int32),
            pltpu.VMEM((CM,), jnp.int32),
            pltpu.VMEM((CM,), jnp.float32),
            pltpu.VMEM((CM, FM), jnp.float32),
            pltpu.VMEM_SHARED((NP, FM), jnp.float32),
            pltpu.SemaphoreType.DMA,
            pltpu.SemaphoreType.DMA,
            pltpu.SemaphoreType.DMA,
            pltpu.SemaphoreType.DMA,
        ],
    )
    def k(y_h, src_h, dst_h, ew_h, msgp_h,
          sidx0, didx0, ewf0, rows0, sidx1, didx1, ewf1, rows1,
          acc_sh, sem_i0, sem_i1, sem_g0, sem_g1):
        cid = lax.axis_index("c")
        sid = lax.axis_index("s")
        wid = cid * 16 + sid
        bufs = ((sidx0, didx0, ewf0, rows0, sem_i0, sem_g0),
                (sidx1, didx1, ewf1, rows1, sem_i1, sem_g1))

        def idx_start(c, b):
            sidx, didx, ewf, _, sem_i, _ = bufs[b]
            eb = wid * PT + c * CM
            pltpu.async_copy(src_h.at[pl.ds(eb, CM)], sidx, sem_i)
            pltpu.async_copy(dst_h.at[pl.ds(eb, CM)], didx, sem_i)
            pltpu.async_copy(ew_h.at[pl.ds(eb, CM)], ewf, sem_i)

        def idx_drain(b):
            sidx, didx, ewf, _, sem_i, _ = bufs[b]
            pltpu.make_async_copy(src_h.at[pl.ds(0, CM)], sidx, sem_i).wait()
            pltpu.make_async_copy(dst_h.at[pl.ds(0, CM)], didx, sem_i).wait()
            pltpu.make_async_copy(ew_h.at[pl.ds(0, CM)], ewf, sem_i).wait()

        def gather_start(b):
            sidx, _, _, rows, _, sem_g = bufs[b]
            pltpu.async_copy(y_h.at[sidx], rows, sem_g)

        def gather_drain(b):
            sidx, _, _, rows, _, sem_g = bufs[b]
            pltpu.make_async_copy(y_h.at[sidx], rows, sem_g).wait()

        def scale(b):
            _, _, ewf, rows, _, _ = bufs[b]

            def body(j, carry2):
                w16 = ewf[pl.ds(j * 16, 16)]
                for i in range(16):
                    e = j * 16 + i
                    w = jnp.full((16,), w16[i])
                    for kk in range(FM // 16):
                        rows[e, pl.ds(kk * 16, 16)] = (
                            rows[e, pl.ds(kk * 16, 16)] * w)
                return carry2

            lax.fori_loop(0, CM // 16, body, 0)

        def scatter(b):
            _, didx, _, rows, _, _ = bufs[b]
            pltpu.sync_copy(rows, acc_sh.at[didx], add=True)

        def zrow(r, carry):
            for kk in range(FM // 16):
                rows0[r, pl.ds(kk * 16, 16)] = jnp.zeros((16,), jnp.float32)
            return carry

        lax.fori_loop(0, CM, zrow, 0)
        for p in range(RPT // CM):
            pltpu.sync_copy(rows0,
                            acc_sh.at[pl.ds(sid * RPT + p * CM, CM)])
        plsc.subcore_barrier()

        idx_start(0, 0)
        idx_drain(0)
        gather_start(0)
        idx_start(1, 1)

        def step(i, carry):
            c0 = 2 * i
            c1 = c0 + 1
            idx_drain(1)
            gather_start(1)
            gather_drain(0)
            scale(0)
            scatter(0)

            @pl.when(c0 + 2 < NCM)
            def _():
                idx_start(c0 + 2, 0)

            @pl.when(c1 + 1 < NCM)
            def _():
                idx_drain(0)
                gather_start(0)

            gather_drain(1)
            scale(1)
            scatter(1)

            @pl.when(c1 + 2 < NCM)
            def _():
                idx_start(c1 + 2, 1)

            return carry

        lax.fori_loop(0, NCM // 2, step, 0)
        plsc.subcore_barrier()
        for p in range(RPT // CM):
            pltpu.sync_copy(acc_sh.at[pl.ds(sid * RPT + p * CM, CM)], rows0)
            pltpu.sync_copy(rows0,
                            msgp_h.at[pl.ds(cid * NP + sid * RPT + p * CM,
                                            CM)])

    return k(y, src1d, dst1d, ew1d)


def _tc_prep(xp, dega8, W1s):
    def body(x_ref, deg_ref, w_ref, yf_ref, ys_ref, yg_ref, dis_ref):
        x = x_ref[...]
        dis_cols = []
        for v, y_ref in enumerate((yf_ref, ys_ref, yg_ref)):
            deg = deg_ref[:, 2 * v:2 * v + 1] + deg_ref[:, 2 * v + 1:2 * v + 2] + 1.0
            dis = jnp.where(deg > 0, lax.rsqrt(jnp.where(deg > 0, deg, 1.0)), 0.0)
            xw = jnp.dot(x, w_ref[v], preferred_element_type=jnp.float32)
            y_ref[...] = dis * xw
            dis_cols.append(dis)
        dis_ref[...] = jnp.concatenate(
            dis_cols + [jnp.zeros((RB, 5), jnp.float32)], axis=1)

    return pl.pallas_call(
        body,
        grid=(NB,),
        in_specs=[
            pl.BlockSpec((RB, FM), lambda i: (i, 0)),
            pl.BlockSpec((RB, 8), lambda i: (i, 0)),
            pl.BlockSpec((3, FM, FM), lambda i: (0, 0, 0)),
        ],
        out_specs=[
            pl.BlockSpec((RB, FM), lambda i: (i, 0)),
            pl.BlockSpec((RB, FM), lambda i: (i, 0)),
            pl.BlockSpec((RB, FM), lambda i: (i, 0)),
            pl.BlockSpec((RB, 8), lambda i: (i, 0)),
        ],
        out_shape=[jax.ShapeDtypeStruct((NP, FM), jnp.float32)] * 3
        + [jax.ShapeDtypeStruct((NP, 8), jnp.float32)],
    )(xp, dega8, W1s)


def _tc_combine(msgps, ys, dis8, bs, W2s=None):
    with_w2 = W2s is not None

    def body(*refs):
        (mf, ms, mg, yf, ys_, yg, dis_ref, b_ref), rest = refs[:8], refs[8:]
        if with_w2:
            w2_ref = rest[0]
            h_refs = rest[1:4]
            y2_refs = rest[4:7]
            ps_ref = rest[7]
        else:
            h_refs = rest[0:3]
            y2_refs = None
            ps_ref = rest[3]
        gid = pl.program_id(0)
        gr = lax.broadcasted_iota(jnp.int32, (RB, FM), 0) + gid * RB
        for v, (m_ref, y_ref) in enumerate(((mf, yf), (ms, ys_), (mg, yg))):
            dis = dis_ref[:, v:v + 1]
            t = m_ref[0] + m_ref[1] + y_ref[...]
            h = jnp.maximum(dis * t + b_ref[v][None, :], 0.0)
            h_refs[v][...] = h
            hm = jnp.where(gr < N, h, 0.0)
            s = jnp.sum(hm, axis=0, keepdims=True)
            ps_ref[v, 0] = jnp.concatenate(
                [s, jnp.zeros((7, FM), jnp.float32)], axis=0)
            if with_w2:
                y2_refs[v][...] = dis * jnp.dot(
                    h, w2_ref[v], preferred_element_type=jnp.float32)

    blk = pl.BlockSpec((RB, FM), lambda i: (i, 0))
    mblk = pl.BlockSpec((2, RB, FM), lambda i: (0, i, 0))
    in_specs = [mblk, mblk, mblk, blk, blk, blk,
                pl.BlockSpec((RB, 8), lambda i: (i, 0)),
                pl.BlockSpec((3, FM), lambda i: (0, 0))]
    out_specs = [blk, blk, blk]
    out_shape = [jax.ShapeDtypeStruct((NP, FM), jnp.float32)] * 3
    if with_w2:
        in_specs.append(pl.BlockSpec((3, FM, FM), lambda i: (0, 0, 0)))
        out_specs += [blk, blk, blk]
        out_shape += [jax.ShapeDtypeStruct((NP, FM), jnp.float32)] * 3
    out_specs.append(pl.BlockSpec((3, 1, 8, FM), lambda i: (0, i, 0, 0)))
    out_shape.append(jax.ShapeDtypeStruct((3, NB, 8, FM), jnp.float32))

    args = list(msgps) + list(ys) + [dis8, bs] + ([W2s] if with_w2 else [])
    return pl.pallas_call(
        body, grid=(NB,), in_specs=in_specs, out_specs=out_specs,
        out_shape=out_shape,
    )(*args)


def _tc_att(pt8, fc1_W, fc1_b2, fc2_W, fc2_b2, cnn_w2):
    def body(p_ref, w1_ref, b1_ref, w2_ref, b2_ref, cw_ref, out_ref):
        m = jnp.sum(p_ref[...], axis=0, keepdims=True) / float(N * FM)
        a1 = jnp.maximum(
            jnp.dot(m[:, :6], w1_ref[...], preferred_element_type=jnp.float32)
            + b1_ref[...], 0.0)
        a2 = jax.nn.sigmoid(
            jnp.dot(a1, w2_ref[...], preferred_element_type=jnp.float32)
            + b2_ref[...])
        out_ref[...] = a2 * cw_ref[...]

    return pl.pallas_call(
        body,
        out_shape=jax.ShapeDtypeStruct((1, 6), jnp.float32),
    )(pt8, fc1_W, fc1_b2, fc2_W, fc2_b2, cnn_w2)


def _tc_mix(h1s, h2s, w8):
    def body(h1f, h1s_, h1g, h2f, h2s_, h2g, w_ref, out_ref):
        acc = (h1f[...] * w_ref[0, 0] + h2f[...] * w_ref[0, 1]
               + h1s_[...] * w_ref[0, 2] + h2s_[...] * w_ref[0, 3]
               + h1g[...] * w_ref[0, 4] + h2g[...] * w_ref[0, 5]
               + w_ref[0, 6])
        out_ref[...] = acc

    blk = pl.BlockSpec((RB, FM), lambda i: (i, 0))
    return pl.pallas_call(
        body,
        grid=(NB,),
        in_specs=[blk] * 6 + [pl.BlockSpec(memory_space=pltpu.SMEM)],
        out_specs=blk,
        out_shape=jax.ShapeDtypeStruct((NP, FM), jnp.float32),
    )(*h1s, *h2s, w8)


def kernel(mm_f_data_matrix, mm_f_edges, mm_s_data_matrix, mm_s_edges,
           mm_g_data_matrix, mm_g_edges, x_m, W_f1, b_f1, W_f2, b_f2,
           W_s1, b_s1, W_s2, b_s2, W_g1, b_g1, W_g2, b_g2,
           fc1_W, fc1_b, fc2_W, fc2_b, cnn_w, cnn_b):
    xp = jnp.pad(x_m, ((0, NP - N), (0, 0)))
    pad_src = jnp.zeros((EP - E,), jnp.int32)
    pad_dst = jnp.full((EP - E,), N, jnp.int32)

    srcs, dsts, ew1ds, degs = [], [], [], []
    for M, e in ((mm_f_data_matrix, mm_f_edges),
                 (mm_s_data_matrix, mm_s_edges),
                 (mm_g_data_matrix, mm_g_edges)):
        src1d = jnp.concatenate([e[0], pad_src])
        dst1d = jnp.concatenate([e[1], pad_dst])
        ew2d, degp = _sc_edge_prep(M.reshape(N * N),
                                   src1d.reshape(EP // 128, 128),
                                   dst1d.reshape(EP // 128, 128))
        srcs.append(src1d)
        dsts.append(dst1d)
        ew1ds.append(ew2d.reshape(EP))
        degs.append(degp.reshape(2, NP))

    dega8 = jnp.pad(jnp.concatenate(degs, axis=0).T, ((0, 0), (0, 2)))
    W1s = jnp.stack([W_f1, W_s1, W_g1])
    b1s = jnp.stack([b_f1, b_s1, b_g1])
    W2s = jnp.stack([W_f2, W_s2, W_g2])
    b2s = jnp.stack([b_f2, b_s2, b_g2])

    y1f, y1s, y1g, dis8 = _tc_prep(xp, dega8, W1s)

    msg1 = [_sc_msg(y, srcs[v], dsts[v], ew1ds[v]).reshape(2, NP, FM)
            for v, y in enumerate((y1f, y1s, y1g))]
    h1f, h1s, h1g, y2f, y2s, y2g, ps1 = _tc_combine(
        msg1, (y1f, y1s, y1g), dis8, b1s, W2s)

    msg2 = [_sc_msg(y, srcs[v], dsts[v], ew1ds[v]).reshape(2, NP, FM)
            for v, y in enumerate((y2f, y2s, y2g))]
    h2f, h2s, h2g, ps2 = _tc_combine(msg2, (y2f, y2s, y2g), dis8, b2s)

    pt = jnp.stack([ps1[0], ps2[0], ps1[1], ps2[1], ps1[2], ps2[2]])
    pt8 = jnp.pad(pt.reshape(6, NB * 8 * FM).T, ((0, 0), (0, 2)))
    w6 = _tc_att(pt8, fc1_W, fc1_b.reshape(1, 30), fc2_W,
                 fc2_b.reshape(1, 6), cnn_w.reshape(1, 6))
    w8 = jnp.concatenate(
        [w6, cnn_b.reshape(1, 1), jnp.zeros((1, 1), jnp.float32)], axis=1)

    out = _tc_mix((h1f, h1s, h1g), (h2f, h2s, h2g), w8)
    return out[:N]


# speedup vs baseline: 6.0106x; 1.0001x over previous
import functools

import jax
import jax.numpy as jnp
from jax import lax
from jax.experimental import pallas as pl
from jax.experimental.pallas import tpu as pltpu
from jax.experimental.pallas import tpu_sc as plsc

N = 10000
FM = 128
NP = 10240
E = 320000
EP = 327680
NW = 32
PT = EP // NW
CA = 1024
GA = CA // 128
NCA = PT // CA
CM = 160
NCM = PT // CM
RPT = NP // 16
RB = 2048
NB = NP // RB


def _sc_mesh():
    return plsc.VectorSubcoreMesh(core_axis_name="c", subcore_axis_name="s")


def _sc_edge_prep(m1d, src2d, dst2d):
    @functools.partial(
        pl.kernel,
        mesh=_sc_mesh(),
        out_type=(jax.ShapeDtypeStruct((EP // 128, 128), jnp.float32),
                  jax.ShapeDtypeStruct((2 * NP,), jnp.float32)),
        scratch_types=[
            pltpu.VMEM((GA, 128), jnp.int32),
            pltpu.VMEM((GA, 128), jnp.int32),
            pltpu.VMEM((GA, 128), jnp.int32),
            pltpu.VMEM((GA, 128), jnp.int32),
            pltpu.VMEM((GA, 128), jnp.float32),
            pltpu.VMEM((RPT,), jnp.float32),
            pltpu.VMEM_SHARED((NP,), jnp.float32),
            pltpu.SemaphoreType.DMA,
            pltpu.SemaphoreType.DMA,
        ],
    )
    def k(m_h, src_h, dst_h, ew_h, degp_h,
          sref, dref, fref, dsc, ewb, zbuf, deg_sh, sem_g, sem_sc):
        cid = lax.axis_index("c")
        sid = lax.axis_index("s")
        wid = cid * 16 + sid

        for i in range(RPT // 16):
            zbuf[pl.ds(i * 16, 16)] = jnp.zeros((16,), jnp.float32)
        pltpu.sync_copy(zbuf, deg_sh.at[pl.ds(sid * RPT, RPT)])
        plsc.subcore_barrier()

        def chunk(c, carry):
            gb = wid * (PT // 128) + c * GA
            pltpu.sync_copy(src_h.at[pl.ds(gb, GA)], sref)
            pltpu.sync_copy(dst_h.at[pl.ds(gb, GA)], dref)

            @pl.when(c > 0)
            def _():
                for g in range(GA):
                    pltpu.make_async_copy(
                        ewb.at[g], deg_sh.at[dsc.at[g]], sem_sc).wait()

            for g in range(GA):
                for o in range(8):
                    s16 = sref[g, pl.ds(o * 16, 16)]
                    d16 = dref[g, pl.ds(o * 16, 16)]
                    fref[g, pl.ds(o * 16, 16)] = s16 * N + d16
                    dsc[g, pl.ds(o * 16, 16)] = d16

            cps = [pltpu.async_copy(m_h.at[fref.at[g]], ewb.at[g], sem_g)
                   for g in range(GA)]
            for cp in cps:
                cp.wait()
            pltpu.sync_copy(ewb, ew_h.at[pl.ds(gb, GA)])
            for g in range(GA):
                pltpu.async_copy(ewb.at[g], deg_sh.at[dsc.at[g]], sem_sc,
                                 add=True)
            return carry

        lax.fori_loop(0, NCA, chunk, 0)
        for g in range(GA):
            pltpu.make_async_copy(ewb.at[g], deg_sh.at[dsc.at[g]],
                                  sem_sc).wait()
        plsc.subcore_barrier()
        pltpu.sync_copy(deg_sh.at[pl.ds(sid * RPT, RPT)], zbuf)
        pltpu.sync_copy(zbuf, degp_h.at[pl.ds(cid * NP + sid * RPT, RPT)])

    return k(m1d, src2d, dst2d)


def _sc_msg(y, src1d, dst1d, ew1d):
    @functools.partial(
        pl.kernel,
        mesh=_sc_mesh(),
        out_type=jax.ShapeDtypeStruct((2 * NP, FM), jnp.float32),
        scratch_types=[
            pltpu.VMEM((CM,), jnp.int32),
            pltpu.VMEM((CM,), jnp.int32),
            pltpu.VMEM((CM,), jnp.float32),
            pltpu.VMEM((CM, FM), jnp.float32),
            pltpu.VMEM((CM,), jnp.int32),
            pltpu.VMEM((CM,), jnp.int32),
            pltpu.VMEM((CM,), jnp.float32),
            pltpu.VMEM((CM, FM), jnp.float32),
            pltpu.VMEM_SHARED((NP, FM), jnp.float32),
            pltpu.SemaphoreType.DMA,
            pltpu.SemaphoreType.DMA,
            pltpu.SemaphoreType.DMA,
            pltpu.SemaphoreType.DMA,
        ],
    )
    def k(y_h, src_h, dst_h, ew_h, msgp_h,
          sidx0, didx0, ewf0, rows0, sidx1, didx1, ewf1, rows1,
          acc_sh, sem_i0, sem_i1, sem_g0, sem_g1):
        cid = lax.axis_index("c")
        sid = lax.axis_index("s")
        wid = cid * 16 + sid
        bufs = ((sidx0, didx0, ewf0, rows0, sem_i0, sem_g0),
                (sidx1, didx1, ewf1, rows1, sem_i1, sem_g1))

        def idx_start(c, b):
            sidx, didx, ewf, _, sem_i, _ = bufs[b]
            eb = wid * PT + c * CM
            pltpu.async_copy(src_h.at[pl.ds(eb, CM)], sidx, sem_i)
            pltpu.async_copy(dst_h.at[pl.ds(eb, CM)], didx, sem_i)
            pltpu.async_copy(ew_h.at[pl.ds(eb, CM)], ewf, sem_i)

        def idx_drain(b):
            sidx, didx, ewf, _, sem_i, _ = bufs[b]
            pltpu.make_async_copy(src_h.at[pl.ds(0, CM)], sidx, sem_i).wait()
            pltpu.make_async_copy(dst_h.at[pl.ds(0, CM)], didx, sem_i).wait()
            pltpu.make_async_copy(ew_h.at[pl.ds(0, CM)], ewf, sem_i).wait()

        def gather_start(b):
            sidx, _, _, rows, _, sem_g = bufs[b]
            pltpu.async_copy(y_h.at[sidx], rows, sem_g)

        def gather_drain(b):
            sidx, _, _, rows, _, sem_g = bufs[b]
            pltpu.make_async_copy(y_h.at[sidx], rows, sem_g).wait()

        def scale(b):
            _, _, ewf, rows, _, _ = bufs[b]

            def body(j, carry2):
                w16 = ewf[pl.ds(j * 16, 16)]
                for i in range(16):
                    e = j * 16 + i
                    w = jnp.full((16,), w16[i])
                    for kk in range(FM // 16):
                        rows[e, pl.ds(kk * 16, 16)] = (
                            rows[e, pl.ds(kk * 16, 16)] * w)
                return carry2

            lax.fori_loop(0, CM // 16, body, 0)

        def scatter(b):
            _, didx, _, rows, _, _ = bufs[b]
            pltpu.sync_copy(rows, acc_sh.at[didx], add=True)

        def zrow(r, carry):
            for kk in range(FM // 16):
                rows0[r, pl.ds(kk * 16, 16)] = jnp.zeros((16,), jnp.float32)
            return carry

        lax.fori_loop(0, CM, zrow, 0)
        for p in range(RPT // CM):
            pltpu.sync_copy(rows0,
                            acc_sh.at[pl.ds(sid * RPT + p * CM, CM)])
        plsc.subcore_barrier()

        idx_start(0, 0)
        idx_drain(0)
        gather_start(0)
        idx_start(1, 1)

        def step(i, carry):
            c0 = 2 * i
            c1 = c0 + 1
            idx_drain(1)
            gather_start(1)
            gather_drain(0)
            scale(0)
            scatter(0)

            @pl.when(c0 + 2 < NCM)
            def _():
                idx_start(c0 + 2, 0)

            @pl.when(c1 + 1 < NCM)
            def _():
                idx_drain(0)
                gather_start(0)

            gather_drain(1)
            scale(1)
            scatter(1)

            @pl.when(c1 + 2 < NCM)
            def _():
                idx_start(c1 + 2, 1)

            return carry

        lax.fori_loop(0, NCM // 2, step, 0)
        plsc.subcore_barrier()
        for p in range(RPT // CM):
            pltpu.sync_copy(acc_sh.at[pl.ds(sid * RPT + p * CM, CM)], rows0)
            pltpu.sync_copy(rows0,
                            msgp_h.at[pl.ds(cid * NP + sid * RPT + p * CM,
                                            CM)])

    return k(y, src1d, dst1d, ew1d)


def _tc_prep(xp, dega8, W1s):
    def body(x_ref, deg_ref, w_ref, yf_ref, ys_ref, yg_ref, dis_ref):
        x = x_ref[...]
        dis_cols = []
        for v, y_ref in enumerate((yf_ref, ys_ref, yg_ref)):
            deg = deg_ref[:, 2 * v:2 * v + 1] + deg_ref[:, 2 * v + 1:2 * v + 2] + 1.0
            dis = jnp.where(deg > 0, lax.rsqrt(jnp.where(deg > 0, deg, 1.0)), 0.0)
            xw = jnp.dot(x, w_ref[v], preferred_element_type=jnp.float32)
            y_ref[...] = dis * xw
            dis_cols.append(dis)
        dis_ref[...] = jnp.concatenate(
            dis_cols + [jnp.zeros((RB, 5), jnp.float32)], axis=1)

    return pl.pallas_call(
        body,
        grid=(NB,),
        in_specs=[
            pl.BlockSpec((RB, FM), lambda i: (i, 0)),
            pl.BlockSpec((RB, 8), lambda i: (i, 0)),
            pl.BlockSpec((3, FM, FM), lambda i: (0, 0, 0)),
        ],
        out_specs=[
            pl.BlockSpec((RB, FM), lambda i: (i, 0)),
            pl.BlockSpec((RB, FM), lambda i: (i, 0)),
            pl.BlockSpec((RB, FM), lambda i: (i, 0)),
            pl.BlockSpec((RB, 8), lambda i: (i, 0)),
        ],
        out_shape=[jax.ShapeDtypeStruct((NP, FM), jnp.float32)] * 3
        + [jax.ShapeDtypeStruct((NP, 8), jnp.float32)],
    )(xp, dega8, W1s)


def _tc_combine(msgps, ys, dis8, bs, W2s=None):
    with_w2 = W2s is not None

    def body(*refs):
        (mf, ms, mg, yf, ys_, yg, dis_ref, b_ref), rest = refs[:8], refs[8:]
        if with_w2:
            w2_ref = rest[0]
            h_refs = rest[1:4]
            y2_refs = rest[4:7]
            ps_ref = rest[7]
        else:
            h_refs = rest[0:3]
            y2_refs = None
            ps_ref = rest[3]
        gid = pl.program_id(0)
        gr = lax.broadcasted_iota(jnp.int32, (RB, FM), 0) + gid * RB
        for v, (m_ref, y_ref) in enumerate(((mf, yf), (ms, ys_), (mg, yg))):
            dis = dis_ref[:, v:v + 1]
            t = m_ref[0] + m_ref[1] + y_ref[...]
            h = jnp.maximum(dis * t + b_ref[v][None, :], 0.0)
            h_refs[v][...] = h
            hm = jnp.where(gr < N, h, 0.0)
            s = jnp.sum(hm, axis=0, keepdims=True)
            ps_ref[v, 0] = jnp.concatenate(
                [s, jnp.zeros((7, FM), jnp.float32)], axis=0)
            if with_w2:
                y2_refs[v][...] = dis * jnp.dot(
                    h, w2_ref[v], preferred_element_type=jnp.float32)

    blk = pl.BlockSpec((RB, FM), lambda i: (i, 0))
    mblk = pl.BlockSpec((2, RB, FM), lambda i: (0, i, 0))
    in_specs = [mblk, mblk, mblk, blk, blk, blk,
                pl.BlockSpec((RB, 8), lambda i: (i, 0)),
                pl.BlockSpec((3, FM), lambda i: (0, 0))]
    out_specs = [blk, blk, blk]
    out_shape = [jax.ShapeDtypeStruct((NP, FM), jnp.float32)] * 3
    if with_w2:
        in_specs.append(pl.BlockSpec((3, FM, FM), lambda i: (0, 0, 0)))
        out_specs += [blk, blk, blk]
        out_shape += [jax.ShapeDtypeStruct((NP, FM), jnp.float32)] * 3
    out_specs.append(pl.BlockSpec((3, 1, 8, FM), lambda i: (0, i, 0, 0)))
    out_shape.append(jax.ShapeDtypeStruct((3, NB, 8, FM), jnp.float32))

    args = list(msgps) + list(ys) + [dis8, bs] + ([W2s] if with_w2 else [])
    return pl.pallas_call(
        body, grid=(NB,), in_specs=in_specs, out_specs=out_specs,
        out_shape=out_shape,
    )(*args)


def _tc_att(pt8, fc1_W, fc1_b2, fc2_W, fc2_b2, cnn_w2):
    def body(p_ref, w1_ref, b1_ref, w2_ref, b2_ref, cw_ref, out_ref):
        m = jnp.sum(p_ref[...], axis=0, keepdims=True) / float(N * FM)
        a1 = jnp.maximum(
            jnp.dot(m[:, :6], w1_ref[...], preferred_element_type=jnp.float32)
            + b1_ref[...], 0.0)
        a2 = jax.nn.sigmoid(
            jnp.dot(a1, w2_ref[...], preferred_element_type=jnp.float32)
            + b2_ref[...])
        out_ref[...] = a2 * cw_ref[...]

    return pl.pallas_call(
        body,
        out_shape=jax.ShapeDtypeStruct((1, 6), jnp.float32),
    )(pt8, fc1_W, fc1_b2, fc2_W, fc2_b2, cnn_w2)


def _tc_mix(h1s, h2s, w8):
    def body(h1f, h1s_, h1g, h2f, h2s_, h2g, w_ref, out_ref):
        acc = (h1f[...] * w_ref[0, 0] + h2f[...] * w_ref[0, 1]
               + h1s_[...] * w_ref[0, 2] + h2s_[...] * w_ref[0, 3]
               + h1g[...] * w_ref[0, 4] + h2g[...] * w_ref[0, 5]
               + w_ref[0, 6])
        out_ref[...] = acc

    blk = pl.BlockSpec((RB, FM), lambda i: (i, 0))
    return pl.pallas_call(
        body,
        grid=(NB,),
        in_specs=[blk] * 6 + [pl.BlockSpec(memory_space=pltpu.SMEM)],
        out_specs=blk,
        out_shape=jax.ShapeDtypeStruct((NP, FM), jnp.float32),
    )(*h1s, *h2s, w8)


def kernel(mm_f_data_matrix, mm_f_edges, mm_s_data_matrix, mm_s_edges,
           mm_g_data_matrix, mm_g_edges, x_m, W_f1, b_f1, W_f2, b_f2,
           W_s1, b_s1, W_s2, b_s2, W_g1, b_g1, W_g2, b_g2,
           fc1_W, fc1_b, fc2_W, fc2_b, cnn_w, cnn_b):
    xp = jnp.pad(x_m, ((0, NP - N), (0, 0)))
    pad_src = jnp.zeros((EP - E,), jnp.int32)
    pad_dst = jnp.full((EP - E,), N, jnp.int32)

    srcs, dsts, ew1ds, degs = [], [], [], []
    for M, e in ((mm_f_data_matrix, mm_f_edges),
                 (mm_s_data_matrix, mm_s_edges),
                 (mm_g_data_matrix, mm_g_edges)):
        src1d = jnp.concatenate([e[0], pad_src])
        dst1d = jnp.concatenate([e[1], pad_dst])
        ew2d, degp = _sc_edge_prep(M.reshape(N * N),
                                   src1d.reshape(EP // 128, 128),
                                   dst1d.reshape(EP // 128, 128))
        srcs.append(src1d)
        dsts.append(dst1d)
        ew1ds.append(ew2d.reshape(EP))
        degs.append(degp.reshape(2, NP))

    dega8 = jnp.pad(jnp.concatenate(degs, axis=0).T, ((0, 0), (0, 2)))
    W1s = jnp.stack([W_f1, W_s1, W_g1])
    b1s = jnp.stack([b_f1, b_s1, b_g1])
    W2s = jnp.stack([W_f2, W_s2, W_g2])
    b2s = jnp.stack([b_f2, b_s2, b_g2])

    y1f, y1s, y1g, dis8 = _tc_prep(xp, dega8, W1s)

    msg1 = [_sc_msg(y, srcs[v], dsts[v], ew1ds[v]).reshape(2, NP, FM)
            for v, y in enumerate((y1f, y1s, y1g))]
    h1f, h1s, h1g, y2f, y2s, y2g, ps1 = _tc_combine(
        msg1, (y1f, y1s, y1g), dis8, b1s, W2s)

    msg2 = [_sc_msg(y, srcs[v], dsts[v], ew1ds[v]).reshape(2, NP, FM)
            for v, y in enumerate((y2f, y2s, y2g))]
    h2f, h2s, h2g, ps2 = _tc_combine(msg2, (y2f, y2s, y2g), dis8, b2s)

    pt = jnp.stack([ps1[0], ps2[0], ps1[1], ps2[1], ps1[2], ps2[2]])
    pt8 = jnp.pad(pt.reshape(6, NB * 8 * FM).T, ((0, 0), (0, 2)))
    w6 = _tc_att(pt8, fc1_W, fc1_b.reshape(1, 30), fc2_W,
                 fc2_b.reshape(1, 6), cnn_w.reshape(1, 6))
    w8 = jnp.concatenate(
        [w6, cnn_b.reshape(1, 1), jnp.zeros((1, 1), jnp.float32)], axis=1)

    out = _tc_mix((h1f, h1s, h1g), (h2f, h2s, h2g), w8)
    return out[:N]
